# baseline (speedup 1.0000x reference)
import jax
import jax.numpy as jnp
from jax.experimental import pallas as pl
from jax.experimental.pallas import tpu as pltpu

_B = 1024
_C, _H, _W = 4, 64, 64
_NT = 1000
_BB = 32


def _body(t_ref, a_ref, b_ref, x_ref, n_ref, o_ref):
    base = pl.program_id(0) * _BB
    for r in range(_BB):
        t_r = t_ref[base + r]
        av = a_ref[t_r]
        bv = b_ref[t_r]
        o_ref[r] = av * x_ref[r] + bv * n_ref[r]


def kernel(x_0, timestep, noise, a, b):
    grid = (_B // _BB,)
    out = pl.pallas_call(
        _body,
        grid=grid,
        in_specs=[
            pl.BlockSpec((_B,), lambda i: (0,), memory_space=pltpu.SMEM),
            pl.BlockSpec((_NT,), lambda i: (0,), memory_space=pltpu.SMEM),
            pl.BlockSpec((_NT,), lambda i: (0,), memory_space=pltpu.SMEM),
            pl.BlockSpec((_BB, _C, _H, _W), lambda i: (i, 0, 0, 0)),
            pl.BlockSpec((_BB, _C, _H, _W), lambda i: (i, 0, 0, 0)),
        ],
        out_specs=pl.BlockSpec((_BB, _C, _H, _W), lambda i: (i, 0, 0, 0)),
        out_shape=jax.ShapeDtypeStruct((_B, _C, _H, _W), jnp.float32),
        compiler_params=pltpu.CompilerParams(
            dimension_semantics=("arbitrary",),
        ),
    )(timestep, a, b, x_0, noise)
    return out


# speedup vs baseline: 6.4490x; 6.4490x over previous
import jax
import jax.numpy as jnp
from jax.experimental import pallas as pl
from jax.experimental.pallas import tpu as pltpu

_B = 1024
_F = 4 * 64 * 64
_TPAD = 1024
_FB = 512


def _body(t_ref, a_ref, b_ref, x_ref, n_ref, o_ref, av_ref, bv_ref):
    @pl.when(pl.program_id(0) == 0)
    def _gather():
        iota = jax.lax.broadcasted_iota(jnp.int32, (_TPAD, _B), 0)
        oh = iota == t_ref[...]
        av_ref[...] = jnp.sum(jnp.where(oh, a_ref[...], 0.0), axis=0,
                              keepdims=True)
        bv_ref[...] = jnp.sum(jnp.where(oh, b_ref[...], 0.0), axis=0,
                              keepdims=True)

    o_ref[...] = av_ref[...] * x_ref[...] + bv_ref[...] * n_ref[...]


def kernel(x_0, timestep, noise, a, b):
    x2 = x_0.transpose(1, 2, 3, 0).reshape(_F, _B)
    n2 = noise.transpose(1, 2, 3, 0).reshape(_F, _B)
    t2 = timestep.reshape(1, _B).astype(jnp.int32)
    ap = jnp.pad(a, (0, _TPAD - a.shape[0])).reshape(_TPAD, 1)
    bp = jnp.pad(b, (0, _TPAD - b.shape[0])).reshape(_TPAD, 1)

    grid = (_F // _FB,)
    out = pl.pallas_call(
        _body,
        grid=grid,
        in_specs=[
            pl.BlockSpec((1, _B), lambda i: (0, 0)),
            pl.BlockSpec((_TPAD, 1), lambda i: (0, 0)),
            pl.BlockSpec((_TPAD, 1), lambda i: (0, 0)),
            pl.BlockSpec((_FB, _B), lambda i: (i, 0)),
            pl.BlockSpec((_FB, _B), lambda i: (i, 0)),
        ],
        out_specs=pl.BlockSpec((_FB, _B), lambda i: (i, 0)),
        out_shape=jax.ShapeDtypeStruct((_F, _B), jnp.float32),
        scratch_shapes=[
            pltpu.VMEM((1, _B), jnp.float32),
            pltpu.VMEM((1, _B), jnp.float32),
        ],
        compiler_params=pltpu.CompilerParams(
            dimension_semantics=("arbitrary",),
        ),
    )(t2, ap, bp, x2, n2)
    return out.reshape(4, 64, 64, _B).transpose(3, 0, 1, 2)


# speedup vs baseline: 6.5756x; 1.0196x over previous
import jax
import jax.numpy as jnp
from jax.experimental import pallas as pl
from jax.experimental.pallas import tpu as pltpu

_B = 1024
_F = 4 * 64 * 64
_TPAD = 1024
_FB = 1024


def _body(t_ref, a_ref, b_ref, x_ref, n_ref, o_ref, av_ref, bv_ref):
    @pl.when(pl.program_id(0) == 0)
    def _gather():
        iota = jax.lax.broadcasted_iota(jnp.int32, (_TPAD, _B), 0)
        oh = iota == t_ref[...]
        av_ref[...] = jnp.sum(jnp.where(oh, a_ref[...], 0.0), axis=0,
                              keepdims=True)
        bv_ref[...] = jnp.sum(jnp.where(oh, b_ref[...], 0.0), axis=0,
                              keepdims=True)

    o_ref[...] = av_ref[...] * x_ref[...] + bv_ref[...] * n_ref[...]


def kernel(x_0, timestep, noise, a, b):
    x2 = x_0.transpose(1, 2, 3, 0).reshape(_F, _B)
    n2 = noise.transpose(1, 2, 3, 0).reshape(_F, _B)
    t2 = timestep.reshape(1, _B).astype(jnp.int32)
    ap = jnp.pad(a, (0, _TPAD - a.shape[0])).reshape(_TPAD, 1)
    bp = jnp.pad(b, (0, _TPAD - b.shape[0])).reshape(_TPAD, 1)

    grid = (_F // _FB,)
    out = pl.pallas_call(
        _body,
        grid=grid,
        in_specs=[
            pl.BlockSpec((1, _B), lambda i: (0, 0)),
            pl.BlockSpec((_TPAD, 1), lambda i: (0, 0)),
            pl.BlockSpec((_TPAD, 1), lambda i: (0, 0)),
            pl.BlockSpec((_FB, _B), lambda i: (i, 0)),
            pl.BlockSpec((_FB, _B), lambda i: (i, 0)),
        ],
        out_specs=pl.BlockSpec((_FB, _B), lambda i: (i, 0)),
        out_shape=jax.ShapeDtypeStruct((_F, _B), jnp.float32),
        scratch_shapes=[
            pltpu.VMEM((1, _B), jnp.float32),
            pltpu.VMEM((1, _B), jnp.float32),
        ],
        compiler_params=pltpu.CompilerParams(
            dimension_semantics=("arbitrary",),
        ),
    )(t2, ap, bp, x2, n2)
    return out.reshape(4, 64, 64, _B).transpose(3, 0, 1, 2)


# speedup vs baseline: 6.5898x; 1.0022x over previous
import jax
import jax.numpy as jnp
from jax.experimental import pallas as pl
from jax.experimental.pallas import tpu as pltpu

_B = 1024
_F = 4 * 64 * 64
_TPAD = 1024
_FB = 2048


def _body(t_ref, a_ref, b_ref, x_ref, n_ref, o_ref, av_ref, bv_ref):
    @pl.when(pl.program_id(0) == 0)
    def _gather():
        iota = jax.lax.broadcasted_iota(jnp.int32, (_TPAD, _B), 0)
        oh = iota == t_ref[...]
        av_ref[...] = jnp.sum(jnp.where(oh, a_ref[...], 0.0), axis=0,
                              keepdims=True)
        bv_ref[...] = jnp.sum(jnp.where(oh, b_ref[...], 0.0), axis=0,
                              keepdims=True)

    o_ref[...] = av_ref[...] * x_ref[...] + bv_ref[...] * n_ref[...]


def kernel(x_0, timestep, noise, a, b):
    x2 = x_0.transpose(1, 2, 3, 0).reshape(_F, _B)
    n2 = noise.transpose(1, 2, 3, 0).reshape(_F, _B)
    t2 = timestep.reshape(1, _B).astype(jnp.int32)
    ap = jnp.pad(a, (0, _TPAD - a.shape[0])).reshape(_TPAD, 1)
    bp = jnp.pad(b, (0, _TPAD - b.shape[0])).reshape(_TPAD, 1)

    grid = (_F // _FB,)
    out = pl.pallas_call(
        _body,
        grid=grid,
        in_specs=[
            pl.BlockSpec((1, _B), lambda i: (0, 0)),
            pl.BlockSpec((_TPAD, 1), lambda i: (0, 0)),
            pl.BlockSpec((_TPAD, 1), lambda i: (0, 0)),
            pl.BlockSpec((_FB, _B), lambda i: (i, 0)),
            pl.BlockSpec((_FB, _B), lambda i: (i, 0)),
        ],
        out_specs=pl.BlockSpec((_FB, _B), lambda i: (i, 0)),
        out_shape=jax.ShapeDtypeStruct((_F, _B), jnp.float32),
        scratch_shapes=[
            pltpu.VMEM((1, _B), jnp.float32),
            pltpu.VMEM((1, _B), jnp.float32),
        ],
        compiler_params=pltpu.CompilerParams(
            dimension_semantics=("arbitrary",),
        ),
    )(t2, ap, bp, x2, n2)
    return out.reshape(4, 64, 64, _B).transpose(3, 0, 1, 2)
